# baseline (speedup 1.0000x reference)
import jax
import jax.numpy as jnp
from jax.experimental import pallas as pl
from jax.experimental.pallas import tpu as pltpu

BATCH = 32
P_TOTAL = 4096
D = 1100
NUM_LEVELS = 256
P_BLK = 512
N_STEPS = P_TOTAL // P_BLK
_BF = jnp.bfloat16


def _enc_kernel(x_ref, pos_ref, vw_ref, out_ref, acc_ref, tot_ref):
    i = pl.program_id(0)

    @pl.when(i == 0)
    def _init():
        acc_ref[...] = jnp.zeros_like(acc_ref)
        tot_ref[...] = jnp.zeros_like(tot_ref)

    pos = pos_ref[...]

    t = jnp.sum((vw_ref[...] < 0).astype(jnp.int32), axis=0)
    t_bf = t.astype(_BF)

    xf = x_ref[...].astype(jnp.float32)
    idx = jnp.clip(jnp.round(xf / 256.0 * 255.0), 0, NUM_LEVELS - 1)
    idx_bf = idx.astype(_BF)

    tot = jnp.zeros((1, D), jnp.float32)
    for c in range(0, P_BLK, 256):
        tot = tot + jnp.sum(pos[c:c + 256], axis=0, keepdims=True).astype(
            jnp.float32)
    tot_ref[...] += tot

    zero = jnp.zeros((), _BF)
    rows = []
    for b in range(BATCH):
        mask = idx_bf[b, :, None] >= t_bf[None, :]
        masked = jnp.where(mask, pos, zero)
        s = jnp.zeros((D,), jnp.float32)
        for c in range(0, P_BLK, 256):
            s = s + jnp.sum(masked[c:c + 256], axis=0).astype(jnp.float32)
        rows.append(s)
    acc_ref[...] += jnp.stack(rows, axis=0)

    @pl.when(i == N_STEPS - 1)
    def _fin():
        hv = 2.0 * acc_ref[...] - tot_ref[...]
        out_ref[...] = jnp.where(hv > 0, 1.0, -1.0).astype(jnp.float32)


def kernel(x, position_weight, value_weight):
    B = x.shape[0]
    x_flat = x.reshape(B, -1)
    pos_bf = position_weight.astype(_BF)
    return pl.pallas_call(
        _enc_kernel,
        grid=(N_STEPS,),
        in_specs=[
            pl.BlockSpec((BATCH, P_BLK), lambda i: (0, i)),
            pl.BlockSpec((P_BLK, D), lambda i: (i, 0)),
            pl.BlockSpec((NUM_LEVELS, D), lambda i: (0, 0)),
        ],
        out_specs=pl.BlockSpec((BATCH, D), lambda i: (0, 0)),
        out_shape=jax.ShapeDtypeStruct((BATCH, D), jnp.float32),
        scratch_shapes=[
            pltpu.VMEM((BATCH, D), jnp.float32),
            pltpu.VMEM((1, D), jnp.float32),
        ],
    )(x_flat, pos_bf, value_weight)


# speedup vs baseline: 1.0823x; 1.0823x over previous
import jax
import jax.numpy as jnp
from jax.experimental import pallas as pl
from jax.experimental.pallas import tpu as pltpu

BATCH = 32
P_TOTAL = 4096
D = 1100
NUM_LEVELS = 256
P_BLK = 512
N_STEPS = P_TOTAL // P_BLK


def _enc_kernel(x_ref, pos_ref, vw_ref, out_ref, acc_ref, tot_ref):
    i = pl.program_id(0)

    @pl.when(i == 0)
    def _init():
        acc_ref[...] = jnp.zeros_like(acc_ref)
        tot_ref[...] = jnp.zeros_like(tot_ref)

    pos = pos_ref[...]

    t = jnp.sum((vw_ref[...] < 0).astype(jnp.int32), axis=0)

    xf = x_ref[...].astype(jnp.float32)
    idx = jnp.round(xf / 256.0 * 255.0)
    idx = jnp.clip(idx, 0, NUM_LEVELS - 1).astype(jnp.int32)

    tot_ref[...] += jnp.sum(pos, axis=0, keepdims=True)

    rows = []
    for b in range(BATCH):
        mask = idx[b, :, None] >= t[None, :]
        masked = jnp.where(mask, pos, 0.0)
        rows.append(jnp.sum(masked, axis=0))
    acc_ref[...] += jnp.stack(rows, axis=0)

    @pl.when(i == N_STEPS - 1)
    def _fin():
        hv = 2.0 * acc_ref[...] - tot_ref[...]
        out_ref[...] = jnp.where(hv > 0, 1.0, -1.0).astype(jnp.float32)


def kernel(x, position_weight, value_weight):
    B = x.shape[0]
    x_flat = x.reshape(B, -1)
    return pl.pallas_call(
        _enc_kernel,
        grid=(N_STEPS,),
        in_specs=[
            pl.BlockSpec((BATCH, P_BLK), lambda i: (0, i)),
            pl.BlockSpec((P_BLK, D), lambda i: (i, 0)),
            pl.BlockSpec((NUM_LEVELS, D), lambda i: (0, 0)),
        ],
        out_specs=pl.BlockSpec((BATCH, D), lambda i: (0, 0)),
        out_shape=jax.ShapeDtypeStruct((BATCH, D), jnp.float32),
        scratch_shapes=[
            pltpu.VMEM((BATCH, D), jnp.float32),
            pltpu.VMEM((1, D), jnp.float32),
        ],
    )(x_flat, position_weight, value_weight)


# speedup vs baseline: 1.0884x; 1.0056x over previous
import jax
import jax.numpy as jnp
from jax.experimental import pallas as pl
from jax.experimental.pallas import tpu as pltpu

BATCH = 32
P_TOTAL = 4096
D = 1100
NUM_LEVELS = 256
P_BLK = 512
N_STEPS = P_TOTAL // P_BLK


def _enc_kernel(x_ref, pos_ref, vw_ref, out_ref, acc_ref, tot_ref, t_ref):
    i = pl.program_id(0)

    @pl.when(i == 0)
    def _init():
        acc_ref[...] = jnp.zeros_like(acc_ref)
        tot_ref[...] = jnp.zeros_like(tot_ref)
        t_ref[...] = jnp.sum((vw_ref[...] < 0).astype(jnp.int32), axis=0,
                             keepdims=True)

    pos = pos_ref[...]
    t = t_ref[0, :]

    xf = x_ref[...].astype(jnp.float32)
    idx = jnp.round(xf / 256.0 * 255.0)
    idx = jnp.clip(idx, 0, NUM_LEVELS - 1).astype(jnp.int32)

    tot_ref[...] += jnp.sum(pos, axis=0, keepdims=True)

    rows = []
    for b in range(BATCH):
        mask = idx[b, :, None] >= t[None, :]
        masked = jnp.where(mask, pos, 0.0)
        rows.append(jnp.sum(masked, axis=0))
    acc_ref[...] += jnp.stack(rows, axis=0)

    @pl.when(i == N_STEPS - 1)
    def _fin():
        hv = 2.0 * acc_ref[...] - tot_ref[...]
        out_ref[...] = jnp.where(hv > 0, 1.0, -1.0).astype(jnp.float32)


def kernel(x, position_weight, value_weight):
    B = x.shape[0]
    x_flat = x.reshape(B, -1)
    return pl.pallas_call(
        _enc_kernel,
        grid=(N_STEPS,),
        in_specs=[
            pl.BlockSpec((BATCH, P_BLK), lambda i: (0, i)),
            pl.BlockSpec((P_BLK, D), lambda i: (i, 0)),
            pl.BlockSpec((NUM_LEVELS, D), lambda i: (0, 0)),
        ],
        out_specs=pl.BlockSpec((BATCH, D), lambda i: (0, 0)),
        out_shape=jax.ShapeDtypeStruct((BATCH, D), jnp.float32),
        scratch_shapes=[
            pltpu.VMEM((BATCH, D), jnp.float32),
            pltpu.VMEM((1, D), jnp.float32),
            pltpu.VMEM((1, D), jnp.int32),
        ],
    )(x_flat, position_weight, value_weight)
